# Initial kernel scaffold; baseline (speedup 1.0000x reference)
import jax
import jax.numpy as jnp
from jax.experimental import pallas as pl


def kernel(inputs, targets):
    raise NotImplementedError("write your pallas kernel here")


# speedup vs baseline: 2.6982x; 2.6982x over previous
import jax
import jax.numpy as jnp
from jax.experimental import pallas as pl
from jax.experimental.pallas import tpu as pltpu

_C = 7
_ALPHA = 0.25
_LS = 0.1
_W = 0.7


def _loss_kernel(x_ref, t_ref, acc_ref):
    j = pl.program_id(1)
    x = x_ref[...]
    t = t_ref[0].astype(jnp.float32)
    c = jax.lax.broadcasted_iota(jnp.int32, x.shape, 0).astype(jnp.float32)

    m = jnp.max(x, axis=0, keepdims=True)
    e = jnp.exp(x - m)
    se = jnp.sum(e, axis=0, keepdims=True)
    sx = jnp.sum(x, axis=0, keepdims=True)
    d = jnp.sum(jnp.abs(c - t) * e, axis=0, keepdims=True)
    xt = jnp.sum(jnp.where(c == t, x, 0.0), axis=0, keepdims=True)

    logz = m + jnp.log(se)
    w = d / se
    pt = jnp.exp(xt - logz)
    ce_r = -((1.0 - _LS) * (xt - logz) + (_LS / _C) * (sx - _C * logz))
    f = 1.0 - pt
    fw = f * f

    part = jnp.concatenate([ce_r, fw, w], axis=0)

    @pl.when(j == 0)
    def _():
        acc_ref[...] = jnp.zeros_like(acc_ref)

    acc_ref[...] = acc_ref[...] + part[None]


def kernel(inputs, targets):
    B, C = inputs.shape
    L = 131072
    if B % (2 * L) != 0:
        L = B // 2
    nblk = B // L
    J = nblk // 2

    x_t = inputs.T
    t3 = targets.astype(jnp.int32).reshape(nblk, 1, L)

    parts = pl.pallas_call(
        _loss_kernel,
        grid=(2, J),
        in_specs=[
            pl.BlockSpec((C, L), lambda i, j: (0, i * J + j)),
            pl.BlockSpec((1, 1, L), lambda i, j: (i * J + j, 0, 0)),
        ],
        out_specs=pl.BlockSpec((1, 3, L), lambda i, j: (i, 0, 0)),
        out_shape=jax.ShapeDtypeStruct((2, 3, L), jnp.float32),
        compiler_params=pltpu.CompilerParams(
            dimension_semantics=("parallel", "arbitrary"),
        ),
    )(x_t, t3)

    s = parts.sum(axis=(0, 2))
    ce = s[0] / B
    focal = _ALPHA * (s[1] / B) * ce
    return focal + _W * (s[2] / B)
